# baseline (speedup 1.0000x reference)
import jax
import jax.numpy as jnp
from jax import lax
from jax.experimental import pallas as pl
from jax.experimental.pallas import tpu as pltpu
from jax.experimental.pallas import tpu_sc as plsc

N_NODES = 10000
N_EDGES = 640000
CH = 128
G = 512
NGROUPS = N_EDGES // G
G1 = 256
S1 = G1 // CH
NG1 = N_EDGES // G1
NW = 32
RB = 400
GRID = N_NODES // RB
N_PAD = 10240
RPT = N_PAD // 16

_f32 = jnp.float32
_i32 = jnp.int32

_SC_PARAMS = pltpu.CompilerParams(
    use_tc_tiling_on_sc=False, needs_layout_passes=False)

_DN1 = lax.GatherDimensionNumbers(
    offset_dims=(), collapsed_slice_dims=(0,), start_index_map=(0,))


def _take16(x, idx):
    return lax.gather(x, idx[:, None], _DN1, (1,),
                      mode=lax.GatherScatterMode.PROMISE_IN_BOUNDS)


def _tc_a(x_ref, wa_ref, t1_ref, ad_ref, cm_ref):
    t = jnp.dot(x_ref[...], wa_ref[...], preferred_element_type=_f32)
    t1_ref[...] = t
    ad_ref[...] = t[:, 72:80]
    ms = jnp.max(t[:, 64:72], axis=0, keepdims=True)
    md = jnp.max(t[:, 72:80], axis=0, keepdims=True)
    new = jnp.concatenate([ms, md], axis=0)

    @pl.when(pl.program_id(0) == 0)
    def _():
        cm_ref[...] = jnp.full((2, 8), -1e30, _f32)

    cm_ref[...] = jnp.maximum(cm_ref[...], new)


def _call_a(x, wa):
    return pl.pallas_call(
        _tc_a,
        grid=(GRID,),
        in_specs=[
            pl.BlockSpec((RB, 1433), lambda i: (i, 0)),
            pl.BlockSpec((1433, 80), lambda i: (0, 0)),
        ],
        out_specs=[
            pl.BlockSpec((RB, 80), lambda i: (i, 0)),
            pl.BlockSpec((RB, 8), lambda i: (i, 0)),
            pl.BlockSpec((2, 8), lambda i: (0, 0)),
        ],
        out_shape=[
            jax.ShapeDtypeStruct((N_NODES, 80), _f32),
            jax.ShapeDtypeStruct((N_NODES, 8), _f32),
            jax.ShapeDtypeStruct((2, 8), _f32),
        ],
    )(x, wa)


def _edge1(t1_hbm, ad_hbm, src_hbm, dst_hbm, cv_hbm, out_hbm,
           src0, dst0, src1, dst1, src2, dst2, ds0, ds1, ds2,
           row0, row1, row2, ad0, ad1, ad2, cv_v, acc,
           is0, is1, is2, gs00, gs01, gs10, gs11, gs20, gs21,
           ss0, ss1, ss2):
    cid = lax.axis_index("c")
    sid = lax.axis_index("s")
    wid = sid * 2 + cid
    iota = lax.iota(_i32, 16)
    h8 = jnp.bitwise_and(iota, 7)
    hpat = lax.shift_right_logical(iota, 3)
    rowpat = jnp.where(iota < 8, 0, 1)
    mask8 = jnp.where(iota < 8, 1.0, 0.0).astype(_f32)
    z16 = jnp.zeros((16,), _f32)

    SRC = [src0, src1, src2]
    DST = [dst0, dst1, dst2]
    DSS = [ds0, ds1, ds2]
    ROW = [row0, row1, row2]
    AD = [ad0, ad1, ad2]
    IS = [is0, is1, is2]
    GS = [(gs00, gs01), (gs10, gs11), (gs20, gs21)]
    SS = [ss0, ss1, ss2]

    pltpu.sync_copy(cv_hbm, cv_v)

    def _zrow(i, c):
        for j in range(5):
            row0[i, pl.ds(j * 16, 16)] = z16
        return c

    lax.fori_loop(0, 128, _zrow, 0)
    base = sid * RPT
    for j in range(5):
        pltpu.sync_copy(row0.at[pl.ds(0, 128)],
                        acc.at[pl.ds(base + j * 128, 128)])
    plsc.subcore_barrier()

    cv = cv_v[...]

    def _row0_of(k):
        return jnp.minimum(wid + NW * k, NG1 - 1) * S1

    def _issue_idx(k, b):
        r0 = _row0_of(k)
        pltpu.async_copy(src_hbm.at[pl.ds(r0, S1)], SRC[b], IS[b])
        pltpu.async_copy(dst_hbm.at[pl.ds(r0, S1)], DST[b], IS[b])

    def _wait_idx(b):
        pltpu.make_async_copy(src_hbm.at[pl.ds(0, S1)], SRC[b], IS[b]).wait()
        pltpu.make_async_copy(dst_hbm.at[pl.ds(0, S1)], DST[b], IS[b]).wait()

    def _issue_gather(b):
        for j in range(S1):
            pltpu.async_copy(t1_hbm.at[SRC[b].at[j]],
                             ROW[b].at[pl.ds(j * CH, CH)], GS[b][0])
            pltpu.async_copy(ad_hbm.at[DST[b].at[j]],
                             AD[b].at[pl.ds(j * CH, CH)], GS[b][1])

    def _wait_gather(b):
        for j in range(S1):
            pltpu.make_async_copy(t1_hbm.at[SRC[b].at[j]],
                                  ROW[b].at[pl.ds(j * CH, CH)],
                                  GS[b][0]).wait()
            pltpu.make_async_copy(ad_hbm.at[DST[b].at[j]],
                                  AD[b].at[pl.ds(j * CH, CH)],
                                  GS[b][1]).wait()

    def _issue_scatter(b):
        for j in range(S1):
            for u in range(CH // 16):
                DSS[b][j, pl.ds(u * 16, 16)] = DST[b][j, pl.ds(u * 16, 16)]
        for j in range(S1):
            pltpu.async_copy(ROW[b].at[pl.ds(j * CH, CH)],
                             acc.at[DSS[b].at[j]], SS[b], add=True)

    def _wait_scatter(b):
        for j in range(S1):
            pltpu.make_async_copy(ROW[b].at[pl.ds(j * CH, CH)],
                                  acc.at[DSS[b].at[j]], SS[b]).wait()

    def _compute(b):
        row_ref = ROW[b]
        ad_ref = AD[b]

        def _edge(p, c2):
            for q in range(2):
                e = p * 4 + q * 2
                rows2 = jnp.full((16,), e, _i32) + rowpat
                a_s = plsc.load_gather(row_ref, [rows2, h8 + 64])
                a_d = plsc.load_gather(ad_ref, [rows2, h8])
                al = a_s + a_d
                al = jnp.maximum(al, 0.2 * al)
                sv2 = jnp.exp(al - cv)
                for u in range(2):
                    eu = e + u
                    row_ref[eu, pl.ds(64, 16)] = (
                        _take16(sv2, u * 8 + h8) * mask8)
                    for j in range(4):
                        sm = _take16(sv2, u * 8 + hpat + 2 * j)
                        hv = row_ref[eu, pl.ds(j * 16, 16)]
                        row_ref[eu, pl.ds(j * 16, 16)] = hv * sm
            return c2

        lax.fori_loop(0, G1 // 4, _edge, 0)

    pltpu.sync_copy(src_hbm.at[pl.ds(_row0_of(0), S1)], src0)
    pltpu.sync_copy(dst_hbm.at[pl.ds(_row0_of(0), S1)], dst0)
    _issue_gather(0)
    _issue_idx(1, 1)

    def _iter(ti, c):
        for ph in range(3):
            g = 3 * ti + ph
            x = ph
            y = (ph + 1) % 3
            z = (ph + 2) % 3
            _wait_gather(x)
            if ph == 2:
                _wait_scatter(y)
            else:
                @pl.when(ti > 0)
                def _(y=y):
                    _wait_scatter(y)
            _wait_idx(y)
            _issue_gather(y)
            _issue_idx(g + 2, z)
            _compute(x)
            _issue_scatter(x)
        return c

    lax.fori_loop(0, 26, _iter, 0)

    _wait_gather(0)
    _wait_scatter(1)
    _wait_scatter(2)

    @pl.when(wid < 4)
    def _():
        _compute(0)
        _issue_scatter(0)
        _wait_scatter(0)

    _wait_idx(1)

    plsc.subcore_barrier()
    pltpu.sync_copy(acc.at[pl.ds(base, RPT)],
                    out_hbm.at[cid, pl.ds(base, RPT)])


def _call_sc1(t1, ad_t, src2, dst2, cv1):
    f = pl.kernel(
        _edge1,
        mesh=plsc.VectorSubcoreMesh(core_axis_name="c", subcore_axis_name="s"),
        compiler_params=_SC_PARAMS,
        out_type=jax.ShapeDtypeStruct((2, N_PAD, 80), _f32),
        scratch_types=[
            pltpu.VMEM((S1, CH), _i32),
            pltpu.VMEM((S1, CH), _i32),
            pltpu.VMEM((S1, CH), _i32),
            pltpu.VMEM((S1, CH), _i32),
            pltpu.VMEM((S1, CH), _i32),
            pltpu.VMEM((S1, CH), _i32),
            pltpu.VMEM((S1, CH), _i32),
            pltpu.VMEM((S1, CH), _i32),
            pltpu.VMEM((S1, CH), _i32),
            pltpu.VMEM((G1, 80), _f32),
            pltpu.VMEM((G1, 80), _f32),
            pltpu.VMEM((G1, 80), _f32),
            pltpu.VMEM((G1, 8), _f32),
            pltpu.VMEM((G1, 8), _f32),
            pltpu.VMEM((G1, 8), _f32),
            pltpu.VMEM((16,), _f32),
            pltpu.VMEM_SHARED((N_PAD, 80), _f32),
        ] + [pltpu.SemaphoreType.DMA] * 12,
    )
    return f(t1, ad_t, src2, dst2, cv1)


def _tc_b(t1_ref, a0_ref, a1_ref, cm_ref, e8_ref, m_ref, b1_ref, r7_ref,
          t2_ref, cm2_ref):
    cm = cm_ref[...]
    c1 = cm[0:1, :] + cm[1:2, :]
    c1 = jnp.maximum(c1, 0.2 * c1)
    tb = t1_ref[...]
    al = tb[:, 64:72] + tb[:, 72:80]
    al = jnp.maximum(al, 0.2 * al)
    sl = jnp.exp(al - c1)
    acc = a0_ref[...] + a1_ref[...]
    den = acc[:, 64:72] + sl
    e8 = e8_ref[...]
    num = acc[:, 0:64] + tb[:, 0:64] * jnp.dot(
        sl, e8, preferred_element_type=_f32)
    dex = jnp.dot(den, e8, preferred_element_type=_f32)
    g = jnp.maximum(num / (dex + 1e-30) + b1_ref[...], 0.0)
    t2 = jnp.dot(g, m_ref[...], preferred_element_type=_f32) + r7_ref[...]
    t2_ref[...] = t2
    ms = jnp.full((1, 8), jnp.max(t2[:, 8:9]), _f32)
    md = jnp.full((1, 8), jnp.max(t2[:, 9:10]), _f32)
    new = jnp.concatenate([ms, md], axis=0)

    @pl.when(pl.program_id(0) == 0)
    def _():
        cm2_ref[...] = jnp.full((2, 8), -1e30, _f32)

    cm2_ref[...] = jnp.maximum(cm2_ref[...], new)


def _call_b(t1, a0, a1, cm1, e8, m, b1r, r7):
    return pl.pallas_call(
        _tc_b,
        grid=(GRID,),
        in_specs=[
            pl.BlockSpec((RB, 80), lambda i: (i, 0)),
            pl.BlockSpec((RB, 80), lambda i: (i, 0)),
            pl.BlockSpec((RB, 80), lambda i: (i, 0)),
            pl.BlockSpec((2, 8), lambda i: (0, 0)),
            pl.BlockSpec((8, 64), lambda i: (0, 0)),
            pl.BlockSpec((64, 16), lambda i: (0, 0)),
            pl.BlockSpec((1, 64), lambda i: (0, 0)),
            pl.BlockSpec((1, 16), lambda i: (0, 0)),
        ],
        out_specs=[
            pl.BlockSpec((RB, 16), lambda i: (i, 0)),
            pl.BlockSpec((2, 8), lambda i: (0, 0)),
        ],
        out_shape=[
            jax.ShapeDtypeStruct((N_NODES, 16), _f32),
            jax.ShapeDtypeStruct((2, 8), _f32),
        ],
    )(t1, a0, a1, cm1, e8, m, b1r, r7)


def _edge2(t2_hbm, src_hbm, dst_hbm, cv_hbm, out_hbm,
           srcA, dstA, srcB, dstB, rsA, rsB, rdA, rdB, cv_v, acc,
           isA, isB, gsA0, gsA1, gsB0, gsB1, ssA, ssB):
    cid = lax.axis_index("c")
    sid = lax.axis_index("s")
    wid = sid * 2 + cid
    iota = lax.iota(_i32, 16)
    mask8 = jnp.where(iota < 8, 1.0, 0.0).astype(_f32)
    col8 = jnp.full((16,), 8, _i32)
    col9 = jnp.full((16,), 9, _i32)
    z16 = jnp.zeros((16,), _f32)

    pltpu.sync_copy(cv_hbm, cv_v)

    def _zrow(i, c):
        rsA[i, pl.ds(0, 16)] = z16
        return c

    lax.fori_loop(0, G, _zrow, 0)
    base = sid * RPT
    pltpu.sync_copy(rsA, acc.at[pl.ds(base, G)])
    pltpu.sync_copy(rsA.at[pl.ds(0, 128)], acc.at[pl.ds(base + G, 128)])
    plsc.subcore_barrier()

    cv = cv_v[...]
    nk = jnp.where(wid < 2, 40, 39)

    def _row0(k):
        return jnp.minimum(wid + NW * k, NGROUPS - 1) * 4

    def _issue_idx(k, s_ref, d_ref, sem):
        r0 = _row0(k)
        pltpu.async_copy(src_hbm.at[pl.ds(r0, 4)], s_ref, sem)
        pltpu.async_copy(dst_hbm.at[pl.ds(r0, 4)], d_ref, sem)

    def _wait_idx(s_ref, d_ref, sem):
        pltpu.make_async_copy(src_hbm.at[pl.ds(0, 4)], s_ref, sem).wait()
        pltpu.make_async_copy(dst_hbm.at[pl.ds(0, 4)], d_ref, sem).wait()

    def _issue_gather(s_ref, d_ref, rs_ref, rd_ref, sem_r, sem_a):
        for j in range(4):
            pltpu.async_copy(t2_hbm.at[s_ref.at[j]],
                             rs_ref.at[pl.ds(j * CH, CH)], sem_r)
            pltpu.async_copy(t2_hbm.at[d_ref.at[j]],
                             rd_ref.at[pl.ds(j * CH, CH)], sem_a)

    def _wait_gather(s_ref, d_ref, rs_ref, rd_ref, sem_r, sem_a):
        for j in range(4):
            pltpu.make_async_copy(t2_hbm.at[s_ref.at[j]],
                                  rs_ref.at[pl.ds(j * CH, CH)], sem_r).wait()
            pltpu.make_async_copy(t2_hbm.at[d_ref.at[j]],
                                  rd_ref.at[pl.ds(j * CH, CH)], sem_a).wait()

    def _scatter(rs_ref, d_ref, sem):
        for j in range(4):
            pltpu.async_copy(rs_ref.at[pl.ds(j * CH, CH)],
                             acc.at[d_ref.at[j]], sem, add=True)
        for j in range(4):
            pltpu.make_async_copy(rs_ref.at[pl.ds(j * CH, CH)],
                                  acc.at[d_ref.at[j]], sem).wait()

    def _compute(rs_ref, rd_ref):
        def _g16(g, c2):
            r16 = g * 16 + iota
            a_s = plsc.load_gather(rs_ref, [r16, col8])
            a_d = plsc.load_gather(rd_ref, [r16, col9])
            al = a_s + a_d
            al = jnp.maximum(al, 0.2 * al)
            sv = jnp.exp(al - cv)
            for u in range(16):
                e = g * 16 + u
                sb = _take16(sv, jnp.full((16,), u, _i32))
                rv = rs_ref[e, pl.ds(0, 16)]
                rs_ref[e, pl.ds(0, 16)] = rv * (sb * mask8)
            return c2

        lax.fori_loop(0, G // 16, _g16, 0)

    pltpu.sync_copy(src_hbm.at[pl.ds(_row0(0), 4)], srcA)
    pltpu.sync_copy(dst_hbm.at[pl.ds(_row0(0), 4)], dstA)
    _issue_gather(srcA, dstA, rsA, rdA, gsA0, gsA1)
    _issue_idx(1, srcB, dstB, isB)

    def _pair(kk, c):
        kA = 2 * kk
        _wait_gather(srcA, dstA, rsA, rdA, gsA0, gsA1)
        _wait_idx(srcB, dstB, isB)
        _issue_gather(srcB, dstB, rsB, rdB, gsB0, gsB1)
        _compute(rsA, rdA)
        _scatter(rsA, dstA, ssA)
        _issue_idx(kA + 2, srcA, dstA, isA)
        _wait_gather(srcB, dstB, rsB, rdB, gsB0, gsB1)
        _wait_idx(srcA, dstA, isA)
        _issue_gather(srcA, dstA, rsA, rdA, gsA0, gsA1)
        _compute(rsB, rdB)
        _scatter(rsB, dstB, ssB)
        _issue_idx(kA + 3, srcB, dstB, isB)
        return c

    lax.fori_loop(0, nk // 2, _pair, 0)

    _wait_gather(srcA, dstA, rsA, rdA, gsA0, gsA1)

    @pl.when(wid >= 2)
    def _():
        _compute(rsA, rdA)
        _scatter(rsA, dstA, ssA)

    _wait_idx(srcB, dstB, isB)

    plsc.subcore_barrier()
    pltpu.sync_copy(acc.at[pl.ds(base, RPT)],
                    out_hbm.at[cid, pl.ds(base, RPT)])


def _call_sc2(t2, src2, dst2, cv2):
    f = pl.kernel(
        _edge2,
        mesh=plsc.VectorSubcoreMesh(core_axis_name="c", subcore_axis_name="s"),
        compiler_params=_SC_PARAMS,
        out_type=jax.ShapeDtypeStruct((2, N_PAD, 16), _f32),
        scratch_types=[
            pltpu.VMEM((4, CH), _i32),
            pltpu.VMEM((4, CH), _i32),
            pltpu.VMEM((4, CH), _i32),
            pltpu.VMEM((4, CH), _i32),
            pltpu.VMEM((G, 16), _f32),
            pltpu.VMEM((G, 16), _f32),
            pltpu.VMEM((G, 16), _f32),
            pltpu.VMEM((G, 16), _f32),
            pltpu.VMEM((16,), _f32),
            pltpu.VMEM_SHARED((N_PAD, 16), _f32),
        ] + [pltpu.SemaphoreType.DMA] * 8,
    )
    return f(t2, src2, dst2, cv2)


def _tc_c(a0_ref, a1_ref, t2_ref, cm2_ref, b2_ref, o_ref):
    c2 = cm2_ref[0, 0] + cm2_ref[1, 0]
    c2 = jnp.maximum(c2, 0.2 * c2)
    t = t2_ref[...]
    al = t[:, 8:9] + t[:, 9:10]
    al = jnp.maximum(al, 0.2 * al)
    sl = jnp.exp(al - c2)
    z = a0_ref[...][:, 0:8] + a1_ref[...][:, 0:8] + t[:, 0:8] * sl
    o = z[:, 0:7] / (z[:, 7:8] + 1e-30) + b2_ref[...][:, 0:7]
    m = jnp.max(o, axis=1, keepdims=True)
    lse = jnp.log(jnp.sum(jnp.exp(o - m), axis=1, keepdims=True)) + m
    o_ref[...] = jnp.concatenate(
        [o - lse, jnp.zeros((RB, 1), _f32)], axis=1)


def _call_c(a0, a1, t2, cm2, b2r):
    return pl.pallas_call(
        _tc_c,
        grid=(GRID,),
        in_specs=[
            pl.BlockSpec((RB, 16), lambda i: (i, 0)),
            pl.BlockSpec((RB, 16), lambda i: (i, 0)),
            pl.BlockSpec((RB, 16), lambda i: (i, 0)),
            pl.BlockSpec((2, 8), lambda i: (0, 0)),
            pl.BlockSpec((1, 8), lambda i: (0, 0)),
        ],
        out_specs=pl.BlockSpec((RB, 8), lambda i: (i, 0)),
        out_shape=jax.ShapeDtypeStruct((N_NODES, 8), _f32),
    )(a0, a1, t2, cm2, b2r)


def kernel(x, edge_index, W1, att_src1, att_dst1, b1, W2, att_src2,
           att_dst2, b2):
    src2 = edge_index[0].reshape(N_EDGES // CH, CH)
    dst2 = edge_index[1].reshape(N_EDGES // CH, CH)

    w1t = W1.T.astype(_f32)
    eye8 = jnp.eye(8, dtype=_f32)
    a_s = (att_src1[0][:, :, None] * eye8[:, None, :]).reshape(64, 8)
    a_d = (att_dst1[0][:, :, None] * eye8[:, None, :]).reshape(64, 8)
    wa = jnp.concatenate([w1t, w1t @ a_s, w1t @ a_d], axis=1)
    e8 = jnp.repeat(eye8, 8, axis=1)
    w2t = W2.T.astype(_f32)
    m = jnp.concatenate([
        w2t,
        jnp.zeros((64, 1), _f32),
        w2t @ att_src2[0, 0][:, None],
        w2t @ att_dst2[0, 0][:, None],
        jnp.zeros((64, 6), _f32),
    ], axis=1)
    r7 = jnp.zeros((1, 16), _f32).at[0, 7].set(1.0)
    b1r = b1.reshape(1, 64)
    b2r = jnp.concatenate([b2, jnp.zeros((1,), _f32)]).reshape(1, 8)

    t1, ad_t, cm1 = _call_a(x, wa)
    c1 = cm1[0] + cm1[1]
    c1 = jnp.maximum(c1, 0.2 * c1)
    cv1 = jnp.concatenate([c1, c1])
    acc1 = _call_sc1(t1, ad_t, src2, dst2, cv1)
    t2, cm2 = _call_b(t1, acc1[0, :N_NODES], acc1[1, :N_NODES],
                      cm1, e8, m, b1r, r7)
    c2s = cm2[0, 0] + cm2[1, 0]
    c2s = jnp.maximum(c2s, 0.2 * c2s)
    cv2 = jnp.full((16,), c2s, _f32)
    acc2 = _call_sc2(t2, src2, dst2, cv2)
    o = _call_c(acc2[0, :N_NODES], acc2[1, :N_NODES], t2, cm2, b2r)
    return o[:, :7]


# speedup vs baseline: 1.0375x; 1.0375x over previous
import jax
import jax.numpy as jnp
from jax import lax
from jax.experimental import pallas as pl
from jax.experimental.pallas import tpu as pltpu
from jax.experimental.pallas import tpu_sc as plsc

N_NODES = 10000
N_EDGES = 640000
CH = 128
G = 512
NGROUPS = N_EDGES // G
G1 = 256
S1 = G1 // CH
NG1 = N_EDGES // G1
NW = 32
RB = 400
GRID = N_NODES // RB
N_PAD = 10240
RPT = N_PAD // 16

_f32 = jnp.float32
_i32 = jnp.int32

_SC_PARAMS = pltpu.CompilerParams(
    use_tc_tiling_on_sc=False, needs_layout_passes=False)

_DN1 = lax.GatherDimensionNumbers(
    offset_dims=(), collapsed_slice_dims=(0,), start_index_map=(0,))


def _take16(x, idx):
    return lax.gather(x, idx[:, None], _DN1, (1,),
                      mode=lax.GatherScatterMode.PROMISE_IN_BOUNDS)


def _tc_a(x_ref, wa_ref, t1_ref, ad_ref, cm_ref):
    t = jnp.dot(x_ref[...], wa_ref[...], preferred_element_type=_f32)
    t1_ref[...] = t
    ad_ref[...] = t[:, 72:80]
    ms = jnp.max(t[:, 64:72], axis=0, keepdims=True)
    md = jnp.max(t[:, 72:80], axis=0, keepdims=True)
    new = jnp.concatenate([ms, md], axis=0)

    @pl.when(pl.program_id(0) == 0)
    def _():
        cm_ref[...] = jnp.full((2, 8), -1e30, _f32)

    cm_ref[...] = jnp.maximum(cm_ref[...], new)


def _call_a(x, wa):
    return pl.pallas_call(
        _tc_a,
        grid=(GRID,),
        in_specs=[
            pl.BlockSpec((RB, 1433), lambda i: (i, 0)),
            pl.BlockSpec((1433, 80), lambda i: (0, 0)),
        ],
        out_specs=[
            pl.BlockSpec((RB, 80), lambda i: (i, 0)),
            pl.BlockSpec((RB, 8), lambda i: (i, 0)),
            pl.BlockSpec((2, 8), lambda i: (0, 0)),
        ],
        out_shape=[
            jax.ShapeDtypeStruct((N_NODES, 80), _f32),
            jax.ShapeDtypeStruct((N_NODES, 8), _f32),
            jax.ShapeDtypeStruct((2, 8), _f32),
        ],
    )(x, wa)


def _edge1(t1_hbm, ad_hbm, src_hbm, dst_hbm, cv_hbm, out_hbm,
           src0, dst0, src1, dst1, src2, dst2, ds0, ds1, ds2,
           row0, row1, row2, ad0, ad1, ad2, cv_v, acc,
           is0, is1, is2, gs00, gs01, gs10, gs11, gs20, gs21,
           ss0, ss1, ss2):
    cid = lax.axis_index("c")
    sid = lax.axis_index("s")
    wid = sid * 2 + cid
    iota = lax.iota(_i32, 16)
    h8 = jnp.bitwise_and(iota, 7)
    hpat = lax.shift_right_logical(iota, 3)
    rowpat = jnp.where(iota < 8, 0, 1)
    mask8 = jnp.where(iota < 8, 1.0, 0.0).astype(_f32)
    z16 = jnp.zeros((16,), _f32)

    SRC = [src0, src1, src2]
    DST = [dst0, dst1, dst2]
    DSS = [ds0, ds1, ds2]
    ROW = [row0, row1, row2]
    AD = [ad0, ad1, ad2]
    IS = [is0, is1, is2]
    GS = [(gs00, gs01), (gs10, gs11), (gs20, gs21)]
    SS = [ss0, ss1, ss2]

    pltpu.sync_copy(cv_hbm, cv_v)

    def _zrow(i, c):
        for j in range(5):
            row0[i, pl.ds(j * 16, 16)] = z16
        return c

    lax.fori_loop(0, 128, _zrow, 0)
    base = sid * RPT
    for j in range(5):
        pltpu.sync_copy(row0.at[pl.ds(0, 128)],
                        acc.at[pl.ds(base + j * 128, 128)])
    plsc.subcore_barrier()

    cv = cv_v[...]

    def _row0_of(k):
        return jnp.minimum(wid + NW * k, NG1 - 1) * S1

    def _issue_idx(k, b):
        r0 = _row0_of(k)
        pltpu.async_copy(src_hbm.at[pl.ds(r0, S1)], SRC[b], IS[b])
        pltpu.async_copy(dst_hbm.at[pl.ds(r0, S1)], DST[b], IS[b])

    def _wait_idx(b):
        pltpu.make_async_copy(src_hbm.at[pl.ds(0, S1)], SRC[b], IS[b]).wait()
        pltpu.make_async_copy(dst_hbm.at[pl.ds(0, S1)], DST[b], IS[b]).wait()

    def _issue_gather(b):
        for j in range(S1):
            pltpu.async_copy(t1_hbm.at[SRC[b].at[j]],
                             ROW[b].at[pl.ds(j * CH, CH)], GS[b][0])
            pltpu.async_copy(ad_hbm.at[DST[b].at[j]],
                             AD[b].at[pl.ds(j * CH, CH)], GS[b][1])

    def _wait_gather(b):
        for j in range(S1):
            pltpu.make_async_copy(t1_hbm.at[SRC[b].at[j]],
                                  ROW[b].at[pl.ds(j * CH, CH)],
                                  GS[b][0]).wait()
            pltpu.make_async_copy(ad_hbm.at[DST[b].at[j]],
                                  AD[b].at[pl.ds(j * CH, CH)],
                                  GS[b][1]).wait()

    def _issue_scatter(b):
        for j in range(S1):
            for u in range(CH // 16):
                DSS[b][j, pl.ds(u * 16, 16)] = DST[b][j, pl.ds(u * 16, 16)]
        for j in range(S1):
            pltpu.async_copy(ROW[b].at[pl.ds(j * CH, CH)],
                             acc.at[DSS[b].at[j]], SS[b], add=True)

    def _wait_scatter(b):
        for j in range(S1):
            pltpu.make_async_copy(ROW[b].at[pl.ds(j * CH, CH)],
                                  acc.at[DSS[b].at[j]], SS[b]).wait()

    def _compute(b):
        row_ref = ROW[b]
        ad_ref = AD[b]

        def _edge(p, c2):
            e = p * 2
            rows2 = jnp.full((16,), e, _i32) + rowpat
            a_s = plsc.load_gather(row_ref, [rows2, h8 + 64])
            a_d = plsc.load_gather(ad_ref, [rows2, h8])
            al = a_s + a_d
            al = jnp.maximum(al, 0.2 * al)
            sv2 = jnp.exp(al - cv)
            for u in range(2):
                eu = e + u
                row_ref[eu, pl.ds(64, 16)] = _take16(sv2, u * 8 + h8) * mask8
                for j in range(4):
                    sm = _take16(sv2, u * 8 + hpat + 2 * j)
                    hv = row_ref[eu, pl.ds(j * 16, 16)]
                    row_ref[eu, pl.ds(j * 16, 16)] = hv * sm
            return c2

        lax.fori_loop(0, G1 // 2, _edge, 0)

    pltpu.sync_copy(src_hbm.at[pl.ds(_row0_of(0), S1)], src0)
    pltpu.sync_copy(dst_hbm.at[pl.ds(_row0_of(0), S1)], dst0)
    _issue_gather(0)
    _issue_idx(1, 1)

    def _iter(ti, c):
        for ph in range(3):
            g = 3 * ti + ph
            x = ph
            y = (ph + 1) % 3
            z = (ph + 2) % 3
            _wait_gather(x)
            if ph == 2:
                _wait_scatter(y)
            else:
                @pl.when(ti > 0)
                def _(y=y):
                    _wait_scatter(y)
            _wait_idx(y)
            _issue_gather(y)
            _issue_idx(g + 2, z)
            _compute(x)
            _issue_scatter(x)
        return c

    lax.fori_loop(0, 26, _iter, 0)

    _wait_gather(0)
    _wait_scatter(1)
    _wait_scatter(2)

    @pl.when(wid < 4)
    def _():
        _compute(0)
        _issue_scatter(0)
        _wait_scatter(0)

    _wait_idx(1)

    plsc.subcore_barrier()
    pltpu.sync_copy(acc.at[pl.ds(base, RPT)],
                    out_hbm.at[cid, pl.ds(base, RPT)])


def _call_sc1(t1, ad_t, src2, dst2, cv1):
    f = pl.kernel(
        _edge1,
        mesh=plsc.VectorSubcoreMesh(core_axis_name="c", subcore_axis_name="s"),
        compiler_params=_SC_PARAMS,
        out_type=jax.ShapeDtypeStruct((2, N_PAD, 80), _f32),
        scratch_types=[
            pltpu.VMEM((S1, CH), _i32),
            pltpu.VMEM((S1, CH), _i32),
            pltpu.VMEM((S1, CH), _i32),
            pltpu.VMEM((S1, CH), _i32),
            pltpu.VMEM((S1, CH), _i32),
            pltpu.VMEM((S1, CH), _i32),
            pltpu.VMEM((S1, CH), _i32),
            pltpu.VMEM((S1, CH), _i32),
            pltpu.VMEM((S1, CH), _i32),
            pltpu.VMEM((G1, 80), _f32),
            pltpu.VMEM((G1, 80), _f32),
            pltpu.VMEM((G1, 80), _f32),
            pltpu.VMEM((G1, 8), _f32),
            pltpu.VMEM((G1, 8), _f32),
            pltpu.VMEM((G1, 8), _f32),
            pltpu.VMEM((16,), _f32),
            pltpu.VMEM_SHARED((N_PAD, 80), _f32),
        ] + [pltpu.SemaphoreType.DMA] * 12,
    )
    return f(t1, ad_t, src2, dst2, cv1)


def _tc_b(t1_ref, a0_ref, a1_ref, cm_ref, e8_ref, m_ref, b1_ref, r7_ref,
          t2_ref, cm2_ref):
    cm = cm_ref[...]
    c1 = cm[0:1, :] + cm[1:2, :]
    c1 = jnp.maximum(c1, 0.2 * c1)
    tb = t1_ref[...]
    al = tb[:, 64:72] + tb[:, 72:80]
    al = jnp.maximum(al, 0.2 * al)
    sl = jnp.exp(al - c1)
    acc = a0_ref[0] + a1_ref[0]
    den = acc[:, 64:72] + sl
    e8 = e8_ref[...]
    num = acc[:, 0:64] + tb[:, 0:64] * jnp.dot(
        sl, e8, preferred_element_type=_f32)
    dex = jnp.dot(den, e8, preferred_element_type=_f32)
    g = jnp.maximum(num / (dex + 1e-30) + b1_ref[...], 0.0)
    t2 = jnp.dot(g, m_ref[...], preferred_element_type=_f32) + r7_ref[...]
    t2_ref[...] = t2
    ms = jnp.full((1, 8), jnp.max(t2[:, 8:9]), _f32)
    md = jnp.full((1, 8), jnp.max(t2[:, 9:10]), _f32)
    new = jnp.concatenate([ms, md], axis=0)

    @pl.when(pl.program_id(0) == 0)
    def _():
        cm2_ref[...] = jnp.full((2, 8), -1e30, _f32)

    cm2_ref[...] = jnp.maximum(cm2_ref[...], new)


def _call_b(t1, a0, a1, cm1, e8, m, b1r, r7):
    return pl.pallas_call(
        _tc_b,
        grid=(GRID,),
        in_specs=[
            pl.BlockSpec((RB, 80), lambda i: (i, 0)),
            pl.BlockSpec((1, RB, 80), lambda i: (0, i, 0)),
            pl.BlockSpec((1, RB, 80), lambda i: (1, i, 0)),
            pl.BlockSpec((2, 8), lambda i: (0, 0)),
            pl.BlockSpec((8, 64), lambda i: (0, 0)),
            pl.BlockSpec((64, 16), lambda i: (0, 0)),
            pl.BlockSpec((1, 64), lambda i: (0, 0)),
            pl.BlockSpec((1, 16), lambda i: (0, 0)),
        ],
        out_specs=[
            pl.BlockSpec((RB, 16), lambda i: (i, 0)),
            pl.BlockSpec((2, 8), lambda i: (0, 0)),
        ],
        out_shape=[
            jax.ShapeDtypeStruct((N_NODES, 16), _f32),
            jax.ShapeDtypeStruct((2, 8), _f32),
        ],
    )(t1, a0, a1, cm1, e8, m, b1r, r7)


def _edge2(t2_hbm, src_hbm, dst_hbm, cv_hbm, out_hbm,
           srcA, dstA, srcB, dstB, rsA, rsB, rdA, rdB, cv_v, acc,
           isA, isB, gsA0, gsA1, gsB0, gsB1, ssA, ssB):
    cid = lax.axis_index("c")
    sid = lax.axis_index("s")
    wid = sid * 2 + cid
    iota = lax.iota(_i32, 16)
    mask8 = jnp.where(iota < 8, 1.0, 0.0).astype(_f32)
    col8 = jnp.full((16,), 8, _i32)
    col9 = jnp.full((16,), 9, _i32)
    z16 = jnp.zeros((16,), _f32)

    pltpu.sync_copy(cv_hbm, cv_v)

    def _zrow(i, c):
        rsA[i, pl.ds(0, 16)] = z16
        return c

    lax.fori_loop(0, G, _zrow, 0)
    base = sid * RPT
    pltpu.sync_copy(rsA, acc.at[pl.ds(base, G)])
    pltpu.sync_copy(rsA.at[pl.ds(0, 128)], acc.at[pl.ds(base + G, 128)])
    plsc.subcore_barrier()

    cv = cv_v[...]
    nk = jnp.where(wid < 2, 40, 39)

    def _row0(k):
        return jnp.minimum(wid + NW * k, NGROUPS - 1) * 4

    def _issue_idx(k, s_ref, d_ref, sem):
        r0 = _row0(k)
        pltpu.async_copy(src_hbm.at[pl.ds(r0, 4)], s_ref, sem)
        pltpu.async_copy(dst_hbm.at[pl.ds(r0, 4)], d_ref, sem)

    def _wait_idx(s_ref, d_ref, sem):
        pltpu.make_async_copy(src_hbm.at[pl.ds(0, 4)], s_ref, sem).wait()
        pltpu.make_async_copy(dst_hbm.at[pl.ds(0, 4)], d_ref, sem).wait()

    def _issue_gather(s_ref, d_ref, rs_ref, rd_ref, sem_r, sem_a):
        for j in range(4):
            pltpu.async_copy(t2_hbm.at[s_ref.at[j]],
                             rs_ref.at[pl.ds(j * CH, CH)], sem_r)
            pltpu.async_copy(t2_hbm.at[d_ref.at[j]],
                             rd_ref.at[pl.ds(j * CH, CH)], sem_a)

    def _wait_gather(s_ref, d_ref, rs_ref, rd_ref, sem_r, sem_a):
        for j in range(4):
            pltpu.make_async_copy(t2_hbm.at[s_ref.at[j]],
                                  rs_ref.at[pl.ds(j * CH, CH)], sem_r).wait()
            pltpu.make_async_copy(t2_hbm.at[d_ref.at[j]],
                                  rd_ref.at[pl.ds(j * CH, CH)], sem_a).wait()

    def _scatter(rs_ref, d_ref, sem):
        for j in range(4):
            pltpu.async_copy(rs_ref.at[pl.ds(j * CH, CH)],
                             acc.at[d_ref.at[j]], sem, add=True)
        for j in range(4):
            pltpu.make_async_copy(rs_ref.at[pl.ds(j * CH, CH)],
                                  acc.at[d_ref.at[j]], sem).wait()

    def _compute(rs_ref, rd_ref):
        def _g16(g, c2):
            r16 = g * 16 + iota
            a_s = plsc.load_gather(rs_ref, [r16, col8])
            a_d = plsc.load_gather(rd_ref, [r16, col9])
            al = a_s + a_d
            al = jnp.maximum(al, 0.2 * al)
            sv = jnp.exp(al - cv)
            for u in range(16):
                e = g * 16 + u
                sb = _take16(sv, jnp.full((16,), u, _i32))
                rv = rs_ref[e, pl.ds(0, 16)]
                rs_ref[e, pl.ds(0, 16)] = rv * (sb * mask8)
            return c2

        lax.fori_loop(0, G // 16, _g16, 0)

    pltpu.sync_copy(src_hbm.at[pl.ds(_row0(0), 4)], srcA)
    pltpu.sync_copy(dst_hbm.at[pl.ds(_row0(0), 4)], dstA)
    _issue_gather(srcA, dstA, rsA, rdA, gsA0, gsA1)
    _issue_idx(1, srcB, dstB, isB)

    def _pair(kk, c):
        kA = 2 * kk
        _wait_gather(srcA, dstA, rsA, rdA, gsA0, gsA1)
        _wait_idx(srcB, dstB, isB)
        _issue_gather(srcB, dstB, rsB, rdB, gsB0, gsB1)
        _compute(rsA, rdA)
        _scatter(rsA, dstA, ssA)
        _issue_idx(kA + 2, srcA, dstA, isA)
        _wait_gather(srcB, dstB, rsB, rdB, gsB0, gsB1)
        _wait_idx(srcA, dstA, isA)
        _issue_gather(srcA, dstA, rsA, rdA, gsA0, gsA1)
        _compute(rsB, rdB)
        _scatter(rsB, dstB, ssB)
        _issue_idx(kA + 3, srcB, dstB, isB)
        return c

    lax.fori_loop(0, nk // 2, _pair, 0)

    _wait_gather(srcA, dstA, rsA, rdA, gsA0, gsA1)

    @pl.when(wid >= 2)
    def _():
        _compute(rsA, rdA)
        _scatter(rsA, dstA, ssA)

    _wait_idx(srcB, dstB, isB)

    plsc.subcore_barrier()
    pltpu.sync_copy(acc.at[pl.ds(base, RPT)],
                    out_hbm.at[cid, pl.ds(base, RPT)])


def _call_sc2(t2, src2, dst2, cv2):
    f = pl.kernel(
        _edge2,
        mesh=plsc.VectorSubcoreMesh(core_axis_name="c", subcore_axis_name="s"),
        compiler_params=_SC_PARAMS,
        out_type=jax.ShapeDtypeStruct((2, N_PAD, 16), _f32),
        scratch_types=[
            pltpu.VMEM((4, CH), _i32),
            pltpu.VMEM((4, CH), _i32),
            pltpu.VMEM((4, CH), _i32),
            pltpu.VMEM((4, CH), _i32),
            pltpu.VMEM((G, 16), _f32),
            pltpu.VMEM((G, 16), _f32),
            pltpu.VMEM((G, 16), _f32),
            pltpu.VMEM((G, 16), _f32),
            pltpu.VMEM((16,), _f32),
            pltpu.VMEM_SHARED((N_PAD, 16), _f32),
        ] + [pltpu.SemaphoreType.DMA] * 8,
    )
    return f(t2, src2, dst2, cv2)


def _tc_c(a0_ref, a1_ref, t2_ref, cm2_ref, b2_ref, o_ref):
    c2 = cm2_ref[0, 0] + cm2_ref[1, 0]
    c2 = jnp.maximum(c2, 0.2 * c2)
    t = t2_ref[...]
    al = t[:, 8:9] + t[:, 9:10]
    al = jnp.maximum(al, 0.2 * al)
    sl = jnp.exp(al - c2)
    z = a0_ref[0][:, 0:8] + a1_ref[0][:, 0:8] + t[:, 0:8] * sl
    o = z[:, 0:7] / (z[:, 7:8] + 1e-30) + b2_ref[...][:, 0:7]
    m = jnp.max(o, axis=1, keepdims=True)
    lse = jnp.log(jnp.sum(jnp.exp(o - m), axis=1, keepdims=True)) + m
    o_ref[...] = jnp.concatenate(
        [o - lse, jnp.zeros((RB, 1), _f32)], axis=1)


def _call_c(a0, a1, t2, cm2, b2r):
    return pl.pallas_call(
        _tc_c,
        grid=(GRID,),
        in_specs=[
            pl.BlockSpec((1, RB, 16), lambda i: (0, i, 0)),
            pl.BlockSpec((1, RB, 16), lambda i: (1, i, 0)),
            pl.BlockSpec((RB, 16), lambda i: (i, 0)),
            pl.BlockSpec((2, 8), lambda i: (0, 0)),
            pl.BlockSpec((1, 8), lambda i: (0, 0)),
        ],
        out_specs=pl.BlockSpec((RB, 8), lambda i: (i, 0)),
        out_shape=jax.ShapeDtypeStruct((N_NODES, 8), _f32),
    )(a0, a1, t2, cm2, b2r)


def kernel(x, edge_index, W1, att_src1, att_dst1, b1, W2, att_src2,
           att_dst2, b2):
    src2 = edge_index[0].reshape(N_EDGES // CH, CH)
    dst2 = edge_index[1].reshape(N_EDGES // CH, CH)

    w1t = W1.T.astype(_f32)
    eye8 = jnp.eye(8, dtype=_f32)
    a_s = (att_src1[0][:, :, None] * eye8[:, None, :]).reshape(64, 8)
    a_d = (att_dst1[0][:, :, None] * eye8[:, None, :]).reshape(64, 8)
    wa = jnp.concatenate([w1t, w1t @ a_s, w1t @ a_d], axis=1)
    e8 = jnp.repeat(eye8, 8, axis=1)
    w2t = W2.T.astype(_f32)
    m = jnp.concatenate([
        w2t,
        jnp.zeros((64, 1), _f32),
        w2t @ att_src2[0, 0][:, None],
        w2t @ att_dst2[0, 0][:, None],
        jnp.zeros((64, 6), _f32),
    ], axis=1)
    r7 = jnp.zeros((1, 16), _f32).at[0, 7].set(1.0)
    b1r = b1.reshape(1, 64)
    b2r = jnp.concatenate([b2, jnp.zeros((1,), _f32)]).reshape(1, 8)

    t1, ad_t, cm1 = _call_a(x, wa)
    c1 = cm1[0] + cm1[1]
    c1 = jnp.maximum(c1, 0.2 * c1)
    cv1 = jnp.concatenate([c1, c1])
    acc1 = _call_sc1(t1, ad_t, src2, dst2, cv1)
    t2, cm2 = _call_b(t1, acc1, acc1, cm1, e8, m, b1r, r7)
    c2s = cm2[0, 0] + cm2[1, 0]
    c2s = jnp.maximum(c2s, 0.2 * c2s)
    cv2 = jnp.full((16,), c2s, _f32)
    acc2 = _call_sc2(t2, src2, dst2, cv2)
    o = _call_c(acc2, acc2, t2, cm2, b2r)
    return o[:, :7]


# speedup vs baseline: 1.0750x; 1.0361x over previous
import jax
import jax.numpy as jnp
from jax import lax
from jax.experimental import pallas as pl
from jax.experimental.pallas import tpu as pltpu
from jax.experimental.pallas import tpu_sc as plsc

N_NODES = 10000
N_EDGES = 640000
CH = 128
G = 512
NGROUPS = N_EDGES // G
G1 = 256
S1 = G1 // CH
NG1 = N_EDGES // G1
NW = 32
RB = 400
GRID = N_NODES // RB
N_PAD = 10240
RPT = N_PAD // 16

_f32 = jnp.float32
_i32 = jnp.int32

_SC_PARAMS = pltpu.CompilerParams(
    use_tc_tiling_on_sc=False, needs_layout_passes=False)

_DN1 = lax.GatherDimensionNumbers(
    offset_dims=(), collapsed_slice_dims=(0,), start_index_map=(0,))


def _take16(x, idx):
    return lax.gather(x, idx[:, None], _DN1, (1,),
                      mode=lax.GatherScatterMode.PROMISE_IN_BOUNDS)


def _tc_a(x_ref, wa_ref, t1_ref, ad_ref, cm_ref):
    t = jnp.dot(x_ref[...], wa_ref[...], preferred_element_type=_f32)
    t1_ref[...] = t
    ad_ref[...] = t[:, 72:80]
    ms = jnp.max(t[:, 64:72], axis=0, keepdims=True)
    md = jnp.max(t[:, 72:80], axis=0, keepdims=True)
    new = jnp.concatenate([ms, md], axis=0)

    @pl.when(pl.program_id(0) == 0)
    def _():
        cm_ref[...] = jnp.full((2, 8), -1e30, _f32)

    cm_ref[...] = jnp.maximum(cm_ref[...], new)


def _call_a(x, wa):
    return pl.pallas_call(
        _tc_a,
        grid=(GRID,),
        in_specs=[
            pl.BlockSpec((RB, 1433), lambda i: (i, 0)),
            pl.BlockSpec((1433, 80), lambda i: (0, 0)),
        ],
        out_specs=[
            pl.BlockSpec((RB, 80), lambda i: (i, 0)),
            pl.BlockSpec((RB, 8), lambda i: (i, 0)),
            pl.BlockSpec((2, 8), lambda i: (0, 0)),
        ],
        out_shape=[
            jax.ShapeDtypeStruct((N_NODES, 80), _f32),
            jax.ShapeDtypeStruct((N_NODES, 8), _f32),
            jax.ShapeDtypeStruct((2, 8), _f32),
        ],
    )(x, wa)


def _edge1(t1_hbm, ad_hbm, src_hbm, dst_hbm, cv_hbm, out_hbm,
           src0, dst0, src1, dst1, src2, dst2, ds0, ds1, ds2,
           row0, row1, row2, ad0, ad1, ad2, cv_v, acc,
           is0, is1, is2, gs00, gs01, gs10, gs11, gs20, gs21,
           ss0, ss1, ss2):
    cid = lax.axis_index("c")
    sid = lax.axis_index("s")
    wid = sid * 2 + cid
    iota = lax.iota(_i32, 16)
    h8 = jnp.bitwise_and(iota, 7)
    hpat = lax.shift_right_logical(iota, 3)
    rowpat = jnp.where(iota < 8, 0, 1)
    mask8 = jnp.where(iota < 8, 1.0, 0.0).astype(_f32)
    z16 = jnp.zeros((16,), _f32)

    SRC = [src0, src1, src2]
    DST = [dst0, dst1, dst2]
    DSS = [ds0, ds1, ds2]
    ROW = [row0, row1, row2]
    AD = [ad0, ad1, ad2]
    IS = [is0, is1, is2]
    GS = [(gs00, gs01), (gs10, gs11), (gs20, gs21)]
    SS = [ss0, ss1, ss2]

    pltpu.sync_copy(cv_hbm, cv_v)

    def _zrow(i, c):
        for j in range(5):
            row0[i, pl.ds(j * 16, 16)] = z16
        return c

    lax.fori_loop(0, 128, _zrow, 0)
    base = sid * RPT
    for j in range(5):
        pltpu.sync_copy(row0.at[pl.ds(0, 128)],
                        acc.at[pl.ds(base + j * 128, 128)])
    plsc.subcore_barrier()

    cv = cv_v[...]

    def _row0_of(k):
        return jnp.minimum(wid + NW * k, NG1 - 1) * S1

    def _issue_idx(k, b):
        r0 = _row0_of(k)
        pltpu.async_copy(src_hbm.at[pl.ds(r0, S1)], SRC[b], IS[b])
        pltpu.async_copy(dst_hbm.at[pl.ds(r0, S1)], DST[b], IS[b])

    def _wait_idx(b):
        pltpu.make_async_copy(src_hbm.at[pl.ds(0, S1)], SRC[b], IS[b]).wait()
        pltpu.make_async_copy(dst_hbm.at[pl.ds(0, S1)], DST[b], IS[b]).wait()

    def _issue_gather(b):
        for j in range(S1):
            pltpu.async_copy(t1_hbm.at[SRC[b].at[j]],
                             ROW[b].at[pl.ds(j * CH, CH)], GS[b][0])
            pltpu.async_copy(ad_hbm.at[DST[b].at[j]],
                             AD[b].at[pl.ds(j * CH, CH)], GS[b][1])

    def _wait_gather(b):
        for j in range(S1):
            pltpu.make_async_copy(t1_hbm.at[SRC[b].at[j]],
                                  ROW[b].at[pl.ds(j * CH, CH)],
                                  GS[b][0]).wait()
            pltpu.make_async_copy(ad_hbm.at[DST[b].at[j]],
                                  AD[b].at[pl.ds(j * CH, CH)],
                                  GS[b][1]).wait()

    def _issue_scatter(b):
        for j in range(S1):
            for u in range(CH // 16):
                DSS[b][j, pl.ds(u * 16, 16)] = DST[b][j, pl.ds(u * 16, 16)]
        for j in range(S1):
            pltpu.async_copy(ROW[b].at[pl.ds(j * CH, CH)],
                             acc.at[DSS[b].at[j]], SS[b], add=True)

    def _wait_scatter(b):
        for j in range(S1):
            pltpu.make_async_copy(ROW[b].at[pl.ds(j * CH, CH)],
                                  acc.at[DSS[b].at[j]], SS[b]).wait()

    def _compute(b):
        row_ref = ROW[b]
        ad_ref = AD[b]

        def _edge(p, c2):
            e = p * 2
            rows2 = jnp.full((16,), e, _i32) + rowpat
            a_s = plsc.load_gather(row_ref, [rows2, h8 + 64])
            a_d = plsc.load_gather(ad_ref, [rows2, h8])
            al = a_s + a_d
            al = jnp.maximum(al, 0.2 * al)
            sv2 = jnp.exp(al - cv)
            for u in range(2):
                eu = e + u
                row_ref[eu, pl.ds(64, 16)] = _take16(sv2, u * 8 + h8) * mask8
                for j in range(4):
                    sm = _take16(sv2, u * 8 + hpat + 2 * j)
                    hv = row_ref[eu, pl.ds(j * 16, 16)]
                    row_ref[eu, pl.ds(j * 16, 16)] = hv * sm
            return c2

        lax.fori_loop(0, G1 // 2, _edge, 0)

    pltpu.sync_copy(src_hbm.at[pl.ds(_row0_of(0), S1)], src0)
    pltpu.sync_copy(dst_hbm.at[pl.ds(_row0_of(0), S1)], dst0)
    _issue_gather(0)
    _issue_idx(1, 1)

    def _iter(ti, c):
        for ph in range(3):
            g = 3 * ti + ph
            x = ph
            y = (ph + 1) % 3
            z = (ph + 2) % 3
            _wait_gather(x)
            if ph == 2:
                _wait_scatter(y)
            else:
                @pl.when(ti > 0)
                def _(y=y):
                    _wait_scatter(y)
            _wait_idx(y)
            _issue_gather(y)
            _issue_idx(g + 2, z)
            _compute(x)
            _issue_scatter(x)
        return c

    lax.fori_loop(0, 26, _iter, 0)

    _wait_gather(0)
    _wait_scatter(1)
    _wait_scatter(2)

    @pl.when(wid < 4)
    def _():
        _compute(0)
        _issue_scatter(0)
        _wait_scatter(0)

    _wait_idx(1)

    plsc.subcore_barrier()
    pltpu.sync_copy(acc.at[pl.ds(base, RPT)],
                    out_hbm.at[cid, pl.ds(base, RPT)])


def _call_sc1(t1, ad_t, src2, dst2, cv1):
    f = pl.kernel(
        _edge1,
        mesh=plsc.VectorSubcoreMesh(core_axis_name="c", subcore_axis_name="s"),
        compiler_params=_SC_PARAMS,
        out_type=jax.ShapeDtypeStruct((2, N_PAD, 80), _f32),
        scratch_types=[
            pltpu.VMEM((S1, CH), _i32),
            pltpu.VMEM((S1, CH), _i32),
            pltpu.VMEM((S1, CH), _i32),
            pltpu.VMEM((S1, CH), _i32),
            pltpu.VMEM((S1, CH), _i32),
            pltpu.VMEM((S1, CH), _i32),
            pltpu.VMEM((S1, CH), _i32),
            pltpu.VMEM((S1, CH), _i32),
            pltpu.VMEM((S1, CH), _i32),
            pltpu.VMEM((G1, 80), _f32),
            pltpu.VMEM((G1, 80), _f32),
            pltpu.VMEM((G1, 80), _f32),
            pltpu.VMEM((G1, 8), _f32),
            pltpu.VMEM((G1, 8), _f32),
            pltpu.VMEM((G1, 8), _f32),
            pltpu.VMEM((16,), _f32),
            pltpu.VMEM_SHARED((N_PAD, 80), _f32),
        ] + [pltpu.SemaphoreType.DMA] * 12,
    )
    return f(t1, ad_t, src2, dst2, cv1)


def _tc_b(t1_ref, a0_ref, a1_ref, cm_ref, e8_ref, m_ref, b1_ref, r7_ref,
          t2_ref, cm2_ref):
    cm = cm_ref[...]
    c1 = cm[0:1, :] + cm[1:2, :]
    c1 = jnp.maximum(c1, 0.2 * c1)
    tb = t1_ref[...]
    al = tb[:, 64:72] + tb[:, 72:80]
    al = jnp.maximum(al, 0.2 * al)
    sl = jnp.exp(al - c1)
    acc = a0_ref[0] + a1_ref[0]
    den = acc[:, 64:72] + sl
    e8 = e8_ref[...]
    num = acc[:, 0:64] + tb[:, 0:64] * jnp.dot(
        sl, e8, preferred_element_type=_f32)
    dex = jnp.dot(den, e8, preferred_element_type=_f32)
    g = jnp.maximum(num / (dex + 1e-30) + b1_ref[...], 0.0)
    t2 = jnp.dot(g, m_ref[...], preferred_element_type=_f32) + r7_ref[...]
    t2_ref[...] = t2
    ms = jnp.full((1, 8), jnp.max(t2[:, 8:9]), _f32)
    md = jnp.full((1, 8), jnp.max(t2[:, 9:10]), _f32)
    new = jnp.concatenate([ms, md], axis=0)

    @pl.when(pl.program_id(0) == 0)
    def _():
        cm2_ref[...] = jnp.full((2, 8), -1e30, _f32)

    cm2_ref[...] = jnp.maximum(cm2_ref[...], new)


def _call_b(t1, a0, a1, cm1, e8, m, b1r, r7):
    return pl.pallas_call(
        _tc_b,
        grid=(GRID,),
        in_specs=[
            pl.BlockSpec((RB, 80), lambda i: (i, 0)),
            pl.BlockSpec((1, RB, 80), lambda i: (0, i, 0)),
            pl.BlockSpec((1, RB, 80), lambda i: (1, i, 0)),
            pl.BlockSpec((2, 8), lambda i: (0, 0)),
            pl.BlockSpec((8, 64), lambda i: (0, 0)),
            pl.BlockSpec((64, 16), lambda i: (0, 0)),
            pl.BlockSpec((1, 64), lambda i: (0, 0)),
            pl.BlockSpec((1, 16), lambda i: (0, 0)),
        ],
        out_specs=[
            pl.BlockSpec((RB, 16), lambda i: (i, 0)),
            pl.BlockSpec((2, 8), lambda i: (0, 0)),
        ],
        out_shape=[
            jax.ShapeDtypeStruct((N_NODES, 16), _f32),
            jax.ShapeDtypeStruct((2, 8), _f32),
        ],
    )(t1, a0, a1, cm1, e8, m, b1r, r7)


def _edge2(t2_hbm, src_hbm, dst_hbm, cv_hbm, out_hbm,
           src0, dst0, src1, dst1, src2, dst2, ds0, ds1, ds2,
           rs0, rs1, rs2, rd0, rd1, rd2, cv_v, acc,
           is0, is1, is2, gs00, gs01, gs10, gs11, gs20, gs21,
           ss0, ss1, ss2):
    cid = lax.axis_index("c")
    sid = lax.axis_index("s")
    wid = sid * 2 + cid
    iota = lax.iota(_i32, 16)
    mask8 = jnp.where(iota < 8, 1.0, 0.0).astype(_f32)
    col8 = jnp.full((16,), 8, _i32)
    col9 = jnp.full((16,), 9, _i32)
    z16 = jnp.zeros((16,), _f32)
    S2 = G // CH

    SRC = [src0, src1, src2]
    DST = [dst0, dst1, dst2]
    DSS = [ds0, ds1, ds2]
    RS = [rs0, rs1, rs2]
    RD = [rd0, rd1, rd2]
    IS = [is0, is1, is2]
    GS = [(gs00, gs01), (gs10, gs11), (gs20, gs21)]
    SS = [ss0, ss1, ss2]

    pltpu.sync_copy(cv_hbm, cv_v)

    def _zrow(i, c):
        rs0[i, pl.ds(0, 16)] = z16
        return c

    lax.fori_loop(0, 128, _zrow, 0)
    base = sid * RPT
    for j in range(5):
        pltpu.sync_copy(rs0.at[pl.ds(0, 128)],
                        acc.at[pl.ds(base + j * 128, 128)])
    plsc.subcore_barrier()

    cv = cv_v[...]

    def _row0_of(k):
        return jnp.minimum(wid + NW * k, NGROUPS - 1) * S2

    def _issue_idx(k, b):
        r0 = _row0_of(k)
        pltpu.async_copy(src_hbm.at[pl.ds(r0, S2)], SRC[b], IS[b])
        pltpu.async_copy(dst_hbm.at[pl.ds(r0, S2)], DST[b], IS[b])

    def _wait_idx(b):
        pltpu.make_async_copy(src_hbm.at[pl.ds(0, S2)], SRC[b], IS[b]).wait()
        pltpu.make_async_copy(dst_hbm.at[pl.ds(0, S2)], DST[b], IS[b]).wait()

    def _issue_gather(b):
        for j in range(S2):
            pltpu.async_copy(t2_hbm.at[SRC[b].at[j]],
                             RS[b].at[pl.ds(j * CH, CH)], GS[b][0])
            pltpu.async_copy(t2_hbm.at[DST[b].at[j]],
                             RD[b].at[pl.ds(j * CH, CH)], GS[b][1])

    def _wait_gather(b):
        for j in range(S2):
            pltpu.make_async_copy(t2_hbm.at[SRC[b].at[j]],
                                  RS[b].at[pl.ds(j * CH, CH)],
                                  GS[b][0]).wait()
            pltpu.make_async_copy(t2_hbm.at[DST[b].at[j]],
                                  RD[b].at[pl.ds(j * CH, CH)],
                                  GS[b][1]).wait()

    def _issue_scatter(b):
        for j in range(S2):
            for u in range(CH // 16):
                DSS[b][j, pl.ds(u * 16, 16)] = DST[b][j, pl.ds(u * 16, 16)]
        for j in range(S2):
            pltpu.async_copy(RS[b].at[pl.ds(j * CH, CH)],
                             acc.at[DSS[b].at[j]], SS[b], add=True)

    def _wait_scatter(b):
        for j in range(S2):
            pltpu.make_async_copy(RS[b].at[pl.ds(j * CH, CH)],
                                  acc.at[DSS[b].at[j]], SS[b]).wait()

    def _compute(b):
        rs_ref = RS[b]
        rd_ref = RD[b]

        def _g16(g, c2):
            r16 = g * 16 + iota
            a_s = plsc.load_gather(rs_ref, [r16, col8])
            a_d = plsc.load_gather(rd_ref, [r16, col9])
            al = a_s + a_d
            al = jnp.maximum(al, 0.2 * al)
            sv = jnp.exp(al - cv)
            for u in range(16):
                e = g * 16 + u
                sb = _take16(sv, jnp.full((16,), u, _i32))
                rv = rs_ref[e, pl.ds(0, 16)]
                rs_ref[e, pl.ds(0, 16)] = rv * (sb * mask8)
            return c2

        lax.fori_loop(0, G // 16, _g16, 0)

    pltpu.sync_copy(src_hbm.at[pl.ds(_row0_of(0), S2)], src0)
    pltpu.sync_copy(dst_hbm.at[pl.ds(_row0_of(0), S2)], dst0)
    _issue_gather(0)
    _issue_idx(1, 1)

    def _iter(ti, c):
        for ph in range(3):
            g = 3 * ti + ph
            x = ph
            y = (ph + 1) % 3
            z = (ph + 2) % 3
            _wait_gather(x)
            if ph == 2:
                _wait_scatter(y)
            else:
                @pl.when(ti > 0)
                def _(y=y):
                    _wait_scatter(y)
            _wait_idx(y)
            _issue_gather(y)
            _issue_idx(g + 2, z)
            _compute(x)
            _issue_scatter(x)
        return c

    lax.fori_loop(0, 13, _iter, 0)

    _wait_gather(0)
    _wait_scatter(1)
    _wait_scatter(2)

    @pl.when(wid < 2)
    def _():
        _compute(0)
        _issue_scatter(0)
        _wait_scatter(0)

    _wait_idx(1)

    plsc.subcore_barrier()
    pltpu.sync_copy(acc.at[pl.ds(base, RPT)],
                    out_hbm.at[cid, pl.ds(base, RPT)])


def _call_sc2(t2, src2, dst2, cv2):
    S2 = G // CH
    f = pl.kernel(
        _edge2,
        mesh=plsc.VectorSubcoreMesh(core_axis_name="c", subcore_axis_name="s"),
        compiler_params=_SC_PARAMS,
        out_type=jax.ShapeDtypeStruct((2, N_PAD, 16), _f32),
        scratch_types=[
            pltpu.VMEM((S2, CH), _i32),
            pltpu.VMEM((S2, CH), _i32),
            pltpu.VMEM((S2, CH), _i32),
            pltpu.VMEM((S2, CH), _i32),
            pltpu.VMEM((S2, CH), _i32),
            pltpu.VMEM((S2, CH), _i32),
            pltpu.VMEM((S2, CH), _i32),
            pltpu.VMEM((S2, CH), _i32),
            pltpu.VMEM((S2, CH), _i32),
            pltpu.VMEM((G, 16), _f32),
            pltpu.VMEM((G, 16), _f32),
            pltpu.VMEM((G, 16), _f32),
            pltpu.VMEM((G, 16), _f32),
            pltpu.VMEM((G, 16), _f32),
            pltpu.VMEM((G, 16), _f32),
            pltpu.VMEM((16,), _f32),
            pltpu.VMEM_SHARED((N_PAD, 16), _f32),
        ] + [pltpu.SemaphoreType.DMA] * 12,
    )
    return f(t2, src2, dst2, cv2)


def _tc_c(a0_ref, a1_ref, t2_ref, cm2_ref, b2_ref, o_ref):
    c2 = cm2_ref[0, 0] + cm2_ref[1, 0]
    c2 = jnp.maximum(c2, 0.2 * c2)
    t = t2_ref[...]
    al = t[:, 8:9] + t[:, 9:10]
    al = jnp.maximum(al, 0.2 * al)
    sl = jnp.exp(al - c2)
    z = a0_ref[0][:, 0:8] + a1_ref[0][:, 0:8] + t[:, 0:8] * sl
    o = z[:, 0:7] / (z[:, 7:8] + 1e-30) + b2_ref[...][:, 0:7]
    m = jnp.max(o, axis=1, keepdims=True)
    lse = jnp.log(jnp.sum(jnp.exp(o - m), axis=1, keepdims=True)) + m
    o_ref[...] = jnp.concatenate(
        [o - lse, jnp.zeros((RB, 1), _f32)], axis=1)


def _call_c(a0, a1, t2, cm2, b2r):
    return pl.pallas_call(
        _tc_c,
        grid=(GRID,),
        in_specs=[
            pl.BlockSpec((1, RB, 16), lambda i: (0, i, 0)),
            pl.BlockSpec((1, RB, 16), lambda i: (1, i, 0)),
            pl.BlockSpec((RB, 16), lambda i: (i, 0)),
            pl.BlockSpec((2, 8), lambda i: (0, 0)),
            pl.BlockSpec((1, 8), lambda i: (0, 0)),
        ],
        out_specs=pl.BlockSpec((RB, 8), lambda i: (i, 0)),
        out_shape=jax.ShapeDtypeStruct((N_NODES, 8), _f32),
    )(a0, a1, t2, cm2, b2r)


def kernel(x, edge_index, W1, att_src1, att_dst1, b1, W2, att_src2,
           att_dst2, b2):
    src2 = edge_index[0].reshape(N_EDGES // CH, CH)
    dst2 = edge_index[1].reshape(N_EDGES // CH, CH)

    w1t = W1.T.astype(_f32)
    eye8 = jnp.eye(8, dtype=_f32)
    a_s = (att_src1[0][:, :, None] * eye8[:, None, :]).reshape(64, 8)
    a_d = (att_dst1[0][:, :, None] * eye8[:, None, :]).reshape(64, 8)
    wa = jnp.concatenate([w1t, w1t @ a_s, w1t @ a_d], axis=1)
    e8 = jnp.repeat(eye8, 8, axis=1)
    w2t = W2.T.astype(_f32)
    m = jnp.concatenate([
        w2t,
        jnp.zeros((64, 1), _f32),
        w2t @ att_src2[0, 0][:, None],
        w2t @ att_dst2[0, 0][:, None],
        jnp.zeros((64, 6), _f32),
    ], axis=1)
    r7 = jnp.zeros((1, 16), _f32).at[0, 7].set(1.0)
    b1r = b1.reshape(1, 64)
    b2r = jnp.concatenate([b2, jnp.zeros((1,), _f32)]).reshape(1, 8)

    t1, ad_t, cm1 = _call_a(x, wa)
    c1 = cm1[0] + cm1[1]
    c1 = jnp.maximum(c1, 0.2 * c1)
    cv1 = jnp.concatenate([c1, c1])
    acc1 = _call_sc1(t1, ad_t, src2, dst2, cv1)
    t2, cm2 = _call_b(t1, acc1, acc1, cm1, e8, m, b1r, r7)
    c2s = cm2[0, 0] + cm2[1, 0]
    c2s = jnp.maximum(c2s, 0.2 * c2s)
    cv2 = jnp.full((16,), c2s, _f32)
    acc2 = _call_sc2(t2, src2, dst2, cv2)
    o = _call_c(acc2, acc2, t2, cm2, b2r)
    return o[:, :7]
